# baseline (speedup 1.0000x reference)
import jax
import jax.numpy as jnp
from jax.experimental import pallas as pl

_VS = 0.2
_MINX, _MINY, _MINZ = 0.0, -40.0, -3.0
_GY, _GZ = 399.0, 19.0
_MAX_PTS = 32
_MAX_VOX = 2000


def _keys_body(pt_ref, key_ref):
    p = pt_ref[...]
    vs = jnp.float32(_VS)
    ix = jnp.floor_divide(p[0:1, :] - jnp.float32(_MINX), vs)
    iy = jnp.floor_divide(p[1:2, :] - jnp.float32(_MINY), vs)
    iz = jnp.floor_divide(p[2:3, :] - jnp.float32(_MINZ), vs)
    key_ref[...] = ix * (_GY * _GZ) + iy * _GZ + iz


def _decode_body(k_ref, c_ref):
    k = k_ref[...]
    gz = jnp.float32(_GZ)
    gy = jnp.float32(_GY)
    iz = jnp.mod(k, gz)
    iy = jnp.mod(jnp.floor_divide(k, gz), gy)
    ix = jnp.floor_divide(k, gy * gz)
    c_ref[...] = jnp.concatenate([ix, iy, iz], axis=0)


def kernel(points):
    n, d = points.shape
    pt = points.T

    keys = pl.pallas_call(
        _keys_body,
        out_shape=jax.ShapeDtypeStruct((1, n), jnp.float32),
    )(pt).reshape(n)

    uq, inv, cnt = jnp.unique(
        keys, return_inverse=True, return_counts=True, size=n, fill_value=0.0
    )
    inv = inv.reshape(-1)
    _, topi = jax.lax.top_k(cnt, _MAX_VOX)
    mapper = (-jnp.ones(cnt.shape[0], dtype=jnp.int32)).at[topi].set(
        jnp.arange(_MAX_VOX, dtype=jnp.int32)
    )
    zb = mapper[inv]
    order = jnp.argsort(inv)
    starts = jnp.cumsum(cnt) - cnt
    rank_sorted = jnp.arange(inv.shape[0]) - starts[inv[order]]
    rank = jnp.zeros(inv.shape[0], dtype=rank_sorted.dtype).at[order].set(rank_sorted)
    keep = (zb >= 0) & (rank < _MAX_PTS)
    idx_v = jnp.where(keep, zb, _MAX_VOX).astype(jnp.int32)
    idx_s = jnp.where(keep, rank, _MAX_PTS).astype(jnp.int32)
    num_per_voxel = jnp.minimum(cnt[topi], _MAX_PTS)

    sel_keys = uq[topi].reshape(1, _MAX_VOX)
    voxel_coords = pl.pallas_call(
        _decode_body,
        out_shape=jax.ShapeDtypeStruct((3, _MAX_VOX), jnp.float32),
    )(sel_keys).T

    voxels = jnp.zeros((_MAX_VOX + 1, _MAX_PTS + 1, d), dtype=points.dtype)
    voxels = voxels.at[idx_v, idx_s].set(points)[:_MAX_VOX, :_MAX_PTS]
    return voxels, voxel_coords, num_per_voxel
